# baseline (speedup 1.0000x reference)
import functools

import jax
import jax.numpy as jnp
from jax import lax
from jax.experimental import pallas as pl
from jax.experimental.pallas import tpu as pltpu
from jax.experimental.pallas import tpu_sc as plsc

N = 10000
E = 320000
H = 128
C = 40
LANES = 16

NC = 2
NS = 16
NW = NC * NS
EPW = E // NW
CB = 128
NCH = 80
EPP = NCH * CB
RS = 4
ND = N + 8
RPS = 624
RTAIL = N - NS * RPS
RB = 1000

_mesh = plsc.VectorSubcoreMesh(core_axis_name="c", subcore_axis_name="s")
_sc_params = pltpu.CompilerParams(needs_layout_passes=False)


@functools.partial(
    pl.kernel,
    out_type=jax.ShapeDtypeStruct((NC, N, H), jnp.float32),
    mesh=_mesh,
    compiler_params=_sc_params,
    scratch_types=[
        pltpu.VMEM((NCH, CB), jnp.int32),
        pltpu.VMEM((RS, CB), jnp.int32),
        pltpu.VMEM((RS, CB), jnp.int32),
        pltpu.VMEM((RS * CB,), jnp.float32),
        pltpu.VMEM((CB, H), jnp.float32),
        pltpu.VMEM((CB, H), jnp.float32),
        pltpu.VMEM_SHARED((ND, H), jnp.float32),
        pltpu.SemaphoreType.DMA,
        pltpu.SemaphoreType.DMA,
    ],
)
def _msg_kernel(x_hbm, pkd_hbm, ce_hbm, zeros_hbm, part_hbm,
                pkd_v, srci_v, dsti_v, cer_v, rows0_v, rows1_v, agg_sh,
                sem0, sem1):
    c = lax.axis_index("c")
    s = lax.axis_index("s")
    wid = c * NS + s
    rbase = s * RPS
    pltpu.sync_copy(pkd_hbm.at[wid], pkd_v)
    pltpu.sync_copy(zeros_hbm.at[pl.ds(rbase, RPS)],
                    agg_sh.at[pl.ds(rbase, RPS)])

    @pl.when(s == NS - 1)
    def _():
        pltpu.sync_copy(zeros_hbm.at[pl.ds(NS * RPS, RTAIL)],
                        agg_sh.at[pl.ds(NS * RPS, RTAIL)])

    plsc.subcore_barrier()

    def _unpack(ch):
        slot = lax.rem(ch, RS)
        for g in range(CB // LANES):
            sl = pl.ds(g * LANES, LANES)
            p = pkd_v[ch, sl]
            srci_v[slot, sl] = p & 0xFFFF
            dsti_v[slot, sl] = lax.shift_right_logical(p, 16)

    def _issue(ch, buf, sem):
        slot = lax.rem(ch, RS)
        pltpu.async_copy(x_hbm.at[srci_v.at[slot]], buf, sem)
        pltpu.async_copy(ce_hbm.at[wid * NCH + ch],
                         cer_v.at[pl.ds(slot * CB, CB)], sem)

    def _scale_rows(buf, slot):
        def row8(g, carry):
            for k in range(8):
                r = g * 8 + k
                e16 = jnp.full((LANES,), slot * CB + r, jnp.int32)
                w16 = plsc.load_gather(cer_v, [e16])
                for j in range(H // LANES):
                    sl = pl.ds(j * LANES, LANES)
                    buf[r, sl] = buf[r, sl] * w16
            return carry

        lax.fori_loop(0, CB // 8, row8, 0)

    def _step(cur, buf, sem, nxt):
        slot = lax.rem(cur, RS)
        if nxt:
            _unpack(cur + 2)
        pltpu.make_async_copy(x_hbm.at[srci_v.at[slot]], buf, sem).wait()
        pltpu.make_async_copy(ce_hbm.at[wid * NCH + cur],
                              cer_v.at[pl.ds(slot * CB, CB)], sem).wait()
        _scale_rows(buf, slot)
        pltpu.sync_copy(buf, agg_sh.at[dsti_v.at[slot]], add=True)
        if nxt:
            _issue(cur + 2, buf, sem)

    _unpack(0)
    _unpack(1)
    _issue(0, rows0_v, sem0)
    _issue(1, rows1_v, sem1)

    def pair(p, carry):
        ch = p * 2
        _step(ch, rows0_v, sem0, True)
        _step(ch + 1, rows1_v, sem1, True)
        return carry

    lax.fori_loop(0, NCH // 2 - 1, pair, 0)
    _step(NCH - 2, rows0_v, sem0, False)
    _step(NCH - 1, rows1_v, sem1, False)
    plsc.subcore_barrier()
    pltpu.sync_copy(agg_sh.at[pl.ds(rbase, RPS)],
                    part_hbm.at[c, pl.ds(rbase, RPS)])

    @pl.when(s == NS - 1)
    def _():
        pltpu.sync_copy(agg_sh.at[pl.ds(NS * RPS, RTAIL)],
                        part_hbm.at[c, pl.ds(NS * RPS, RTAIL)])


def _tc_layer_body(part_ref, w_ref, b_ref, o_ref):
    t = part_ref[0] + part_ref[1]
    y = jnp.dot(t, w_ref[...], preferred_element_type=jnp.float32) + b_ref[...]
    o_ref[...] = jnp.where(y >= 0, y, 0.01 * y)


_tc_layer = pl.pallas_call(
    _tc_layer_body,
    grid=(N // RB,),
    in_specs=[
        pl.BlockSpec((2, RB, H), lambda i: (0, i, 0)),
        pl.BlockSpec((H, H), lambda i: (0, 0)),
        pl.BlockSpec((1, H), lambda i: (0, 0)),
    ],
    out_specs=pl.BlockSpec((RB, H), lambda i: (i, 0)),
    out_shape=jax.ShapeDtypeStruct((N, H), jnp.float32),
)


def _tc_last_body(part_ref, w6_ref, b6_ref, wl_ref, bl_ref, o_ref):
    t = part_ref[0] + part_ref[1]
    y = jnp.dot(t, w6_ref[...], preferred_element_type=jnp.float32) + b6_ref[...]
    h = jnp.where(y >= 0, y, 0.01 * y)
    o_ref[...] = jnp.dot(h, wl_ref[...],
                         preferred_element_type=jnp.float32) + bl_ref[...]


_tc_last = pl.pallas_call(
    _tc_last_body,
    grid=(N // RB,),
    in_specs=[
        pl.BlockSpec((2, RB, H), lambda i: (0, i, 0)),
        pl.BlockSpec((H, H), lambda i: (0, 0)),
        pl.BlockSpec((1, H), lambda i: (0, 0)),
        pl.BlockSpec((H, C), lambda i: (0, 0)),
        pl.BlockSpec((1, C), lambda i: (0, 0)),
    ],
    out_specs=pl.BlockSpec((RB, C), lambda i: (i, 0)),
    out_shape=jax.ShapeDtypeStruct((N, C), jnp.float32),
)


def kernel(in_feat, edge_index, edge_weight, W1, b1, W2, b2, W3, b3, W4, b4,
           W5, b5, W6, b6, Wl, bl):
    src = edge_index[0]
    dst = edge_index[1]
    deg_out = jnp.clip(jnp.bincount(src, length=N).astype(jnp.float32), 1.0)
    deg_in = jnp.clip(jnp.bincount(dst, length=N).astype(jnp.float32), 1.0)
    do_inv = lax.rsqrt(deg_out)
    di_inv = lax.rsqrt(deg_in)
    ce = edge_weight * do_inv[src] * di_inv[dst]

    pkd = src | (dst << 16)
    pad = EPP - EPW
    pkda = jnp.pad(pkd.reshape(NW, EPW), ((0, 0), (0, pad)),
                   constant_values=N << 16).reshape(NW, NCH, CB)
    cep = jnp.pad(ce.reshape(NW, EPW), ((0, 0), (0, pad))
                  ).reshape(NW * NCH, CB)
    zeros_nh = jnp.zeros((N, H), jnp.float32)

    h = in_feat
    for (W, b) in ((W1, b1), (W2, b2), (W3, b3), (W4, b4), (W5, b5)):
        part = _msg_kernel(h, pkda, cep, zeros_nh)
        h = _tc_layer(part, W, b[None, :])
    part = _msg_kernel(h, pkda, cep, zeros_nh)
    return _tc_last(part, W6, b6[None, :], Wl, bl[None, :])
